# baseline (speedup 1.0000x reference)
import jax
import jax.numpy as jnp
from jax import lax
from jax.experimental import pallas as pl
from jax.experimental.pallas import tpu as pltpu
from jax.experimental.pallas import tpu_sc as plsc

D_MODEL = 64
SCALE = 8.0
SEQ = 50
N_B = 16384
NW = 32

N_TC = 7813
P1S = D_MODEL
N1_ITEMS = 248
NB1 = 4
NO1 = 2

P_B = 256
N_C = N_B // P_B
N_ITEMS = SEQ * 2
NR = 4
NT = 2


def _tr_body(lutt_hbm, tail_hbm, out_hbm, inbs, outbs, tailb, isems, osems):
    wid = lax.axis_index("s") * 2 + lax.axis_index("c")
    lane = lax.iota(jnp.int32, 16)
    md = [jnp.bitwise_and(d0 + lane, 7) for d0 in range(8)]
    sbs = [lane * D_MODEL + md[d0] for d0 in range(8)]

    def tc_of(i):
        return jnp.minimum(wid + NW * i, N_TC - 2)

    def in_start(i, b):
        tc = tc_of(i)
        for band in range(8):
            pltpu.async_copy(
                lutt_hbm.at[pl.ds(8 * band, 8), pl.ds(tc * 128, 128)],
                inbs[b].at[band], isems[b])

    def in_wait(b):
        for band in range(8):
            pltpu.make_async_copy(
                lutt_hbm.at[pl.ds(0, 8), pl.ds(0, 128)],
                inbs[b].at[0], isems[b]).wait()

    def transpose(b, bo):
        inb, outb = inbs[b], outbs[bo]

        @plsc.parallel_loop(0, 64, step=1, unroll=4)
        def _(i):
            band = i // 8
            jv = i % 8
            bb = jnp.full((16,), band, jnp.int32)
            base = 1024 * jv + 8 * band
            vidx = lane + 16 * jv
            for d0 in range(8):
                vv = plsc.load_gather(inb, [bb, md[d0], vidx])
                plsc.store_scatter(outb, [sbs[d0] + base], vv)

    def out_start(i, bo):
        pltpu.async_copy(outbs[bo],
                         out_hbm.at[pl.ds(tc_of(i) * (128 * P1S), 128 * P1S)],
                         osems[bo])

    def out_wait(bo):
        pltpu.make_async_copy(outbs[bo],
                              out_hbm.at[pl.ds(0, 128 * P1S)],
                              osems[bo]).wait()

    def item(i, b, first, last):
        bo = b % NO1
        in_wait(b)
        if not first:
            out_wait(bo)
        transpose(b, bo)
        if not last:
            in_start(i + NB1, b)
        out_start(i, bo)

    for b in range(NB1):
        in_start(b, b)
    for t in range(NB1):
        item(t, t, first=t < NO1, last=False)

    def group(g, carry):
        for b in range(NB1):
            item(g * NB1 + b, b, first=False, last=False)
        return carry

    lax.fori_loop(1, N1_ITEMS // NB1 - 1, group, 0)

    for t in range(N1_ITEMS - NB1, N1_ITEMS):
        item(t, t % NB1, first=False, last=True)
    for bo in range(NO1):
        out_wait(bo)

    @pl.when(wid == 4)
    def _():
        pltpu.sync_copy(tail_hbm, tailb)

        def vrow(v, c):
            for k in range(4):
                vv = tailb[pl.ds(v * 64 + 16 * k, 16)]
                plsc.store_scatter(outbs[0], [v * D_MODEL + 16 * k + lane],
                                   vv)
            return c

        lax.fori_loop(0, 64, vrow, 0)
        pltpu.sync_copy(outbs[0].at[pl.ds(0, 64 * P1S)],
                        out_hbm.at[pl.ds((N_TC - 1) * (128 * P1S),
                                         64 * P1S)])


def _emb_body(xt_hbm, lut_hbm, out_hbm, idxs, rowss, tbufs,
              isems, gsems, wsems):
    wid = lax.axis_index("s") * 2 + lax.axis_index("c")
    lane = lax.iota(jnp.int32, 16)
    dcols = [lane + 16 * k for k in range(4)]

    def idx_start(t, b):
        pltpu.async_copy(xt_hbm.at[t // 2, 2 * wid + t % 2],
                         idxs[b], isems[b])

    def idx_wait(b):
        pltpu.make_async_copy(xt_hbm.at[0, 0], idxs[b], isems[b]).wait()

    def gather_start(b):
        pltpu.async_copy(lut_hbm.at[idxs[b].at[0]],
                         rowss[b].at[pl.ds(0, 128)], gsems[b])
        pltpu.async_copy(lut_hbm.at[idxs[b].at[1]],
                         rowss[b].at[pl.ds(128, 128)], gsems[b])

    def gather_wait(b):
        for _ in range(2):
            pltpu.make_async_copy(lut_hbm.at[idxs[b].at[0]],
                                  rowss[b].at[pl.ds(0, 128)],
                                  gsems[b]).wait()

    def transpose(br, bt):
        rows, tb = rowss[br], tbufs[bt]

        @plsc.parallel_loop(0, P_B, step=1, unroll=4)
        def _(r):
            rb = jnp.full((16,), r, jnp.int32)
            for k in range(4):
                v = rows[r, pl.ds(16 * k, 16)] * SCALE
                plsc.store_scatter(tb, [dcols[k], rb], v)

    def write_start(t, bt):
        c = 2 * wid + t % 2
        pltpu.async_copy(tbufs[bt].at[:, pl.ds(0, P_B)],
                         out_hbm.at[t // 2, :, pl.ds(c * P_B, P_B)],
                         wsems[bt])

    def write_wait(bt):
        pltpu.make_async_copy(tbufs[bt].at[:, pl.ds(0, P_B)],
                              out_hbm.at[0, :, pl.ds(0, P_B)],
                              wsems[bt]).wait()

    def item(t, b, first, last):
        br = b % NR
        bt = b % NT
        gather_wait(br)
        if not last:
            idx_start(t + NR, br)
        if not first:
            write_wait(bt)
        transpose(br, bt)
        if not last:
            idx_wait(br)
            gather_start(br)
        write_start(t, bt)

    for b in range(NR):
        idx_start(b, b)
    for b in range(NR):
        idx_wait(b)
        gather_start(b)

    for t in range(NR):
        item(t, t, first=t < NT, last=False)

    def group(g, carry):
        for b in range(NR):
            item(g * NR + b, b, first=False, last=False)
        return carry

    lax.fori_loop(1, N_ITEMS // NR - 1, group, 0)

    for t in range(N_ITEMS - NR, N_ITEMS):
        item(t, t % NR, first=False, last=True)
    for bt in range(NT):
        write_wait(bt)


def kernel(x, lut):
    B, S = x.shape
    mesh = plsc.VectorSubcoreMesh(core_axis_name="c", subcore_axis_name="s")

    lutp = pl.kernel(
        _tr_body,
        mesh=mesh,
        out_type=jax.ShapeDtypeStruct((N_TC * 128 * P1S,), jnp.float32),
        scratch_types=[
            [pltpu.VMEM((8, 8, 128), jnp.float32) for _ in range(NB1)],
            [pltpu.VMEM((128 * P1S,), jnp.float32) for _ in range(NO1)],
            pltpu.VMEM((64 * D_MODEL,), jnp.float32),
            [pltpu.SemaphoreType.DMA for _ in range(NB1)],
            [pltpu.SemaphoreType.DMA for _ in range(NO1)],
        ],
        compiler_params=pltpu.CompilerParams(
            use_tc_tiling_on_sc=True, needs_layout_passes=False),
    )(lut.T, lut[999936:].reshape(64 * D_MODEL))

    xt = x.T.astype(jnp.int32).reshape(S, N_C, 2, 128)
    out = pl.kernel(
        _emb_body,
        mesh=mesh,
        out_type=jax.ShapeDtypeStruct((SEQ, D_MODEL, N_B), jnp.float32),
        scratch_types=[
            [pltpu.VMEM((2, 128), jnp.int32) for _ in range(NR)],
            [pltpu.VMEM((P_B, D_MODEL), jnp.float32) for _ in range(NR)],
            [pltpu.VMEM((D_MODEL, P_B + 1), jnp.float32) for _ in range(NT)],
            [pltpu.SemaphoreType.DMA for _ in range(NR)],
            [pltpu.SemaphoreType.DMA for _ in range(NR)],
            [pltpu.SemaphoreType.DMA for _ in range(NT)],
        ],
        compiler_params=pltpu.CompilerParams(
            use_tc_tiling_on_sc=False, needs_layout_passes=False),
    )(xt, lutp.reshape(N_TC * 128, D_MODEL))
    return jnp.transpose(out, (2, 0, 1))


# speedup vs baseline: 1.0032x; 1.0032x over previous
import jax
import jax.numpy as jnp
from jax import lax
from jax.experimental import pallas as pl
from jax.experimental.pallas import tpu as pltpu
from jax.experimental.pallas import tpu_sc as plsc

D_MODEL = 64
SCALE = 8.0
SEQ = 50
N_B = 16384
NW = 32

N_TC = 7813
P1S = D_MODEL
N1_ITEMS = 248
NB1 = 4
NO1 = 2

P_B = 256
N_C = N_B // P_B
N_ITEMS = SEQ * 2
NR = 4
NT = 2


def _tr_body(lutt_hbm, tail_hbm, out_hbm, inbs, outbs, tailb, isems, osems):
    wid = lax.axis_index("s") * 2 + lax.axis_index("c")
    lane = lax.iota(jnp.int32, 16)
    md = [jnp.bitwise_and(d0 + lane, 7) for d0 in range(8)]
    sbs = [lane * D_MODEL + md[d0] for d0 in range(8)]

    def tc_of(i):
        return jnp.minimum(wid + NW * i, N_TC - 2)

    def in_start(i, b):
        tc = tc_of(i)
        for band in range(8):
            pltpu.async_copy(
                lutt_hbm.at[pl.ds(8 * band, 8), pl.ds(tc * 128, 128)],
                inbs[b].at[band], isems[b])

    def in_wait(b):
        for band in range(8):
            pltpu.make_async_copy(
                lutt_hbm.at[pl.ds(0, 8), pl.ds(0, 128)],
                inbs[b].at[0], isems[b]).wait()

    def transpose(b, bo):
        inb, outb = inbs[b], outbs[bo]

        @plsc.parallel_loop(0, 64, step=1, unroll=2)
        def _(i):
            band = i // 8
            jv = i % 8
            bb = jnp.full((16,), band, jnp.int32)
            base = 1024 * jv + 8 * band
            vidx = lane + 16 * jv
            for d0 in range(8):
                vv = plsc.load_gather(inb, [bb, md[d0], vidx])
                plsc.store_scatter(outb, [sbs[d0] + base], vv)

    def out_start(i, bo):
        pltpu.async_copy(outbs[bo],
                         out_hbm.at[pl.ds(tc_of(i) * (128 * P1S), 128 * P1S)],
                         osems[bo])

    def out_wait(bo):
        pltpu.make_async_copy(outbs[bo],
                              out_hbm.at[pl.ds(0, 128 * P1S)],
                              osems[bo]).wait()

    def item(i, b, first, last):
        bo = b % NO1
        in_wait(b)
        if not first:
            out_wait(bo)
        transpose(b, bo)
        if not last:
            in_start(i + NB1, b)
        out_start(i, bo)

    for b in range(NB1):
        in_start(b, b)
    for t in range(NB1):
        item(t, t, first=t < NO1, last=False)

    def group(g, carry):
        for b in range(NB1):
            item(g * NB1 + b, b, first=False, last=False)
        return carry

    lax.fori_loop(1, N1_ITEMS // NB1 - 1, group, 0)

    for t in range(N1_ITEMS - NB1, N1_ITEMS):
        item(t, t % NB1, first=False, last=True)
    for bo in range(NO1):
        out_wait(bo)

    @pl.when(wid == 4)
    def _():
        pltpu.sync_copy(tail_hbm, tailb)

        def vrow(v, c):
            for k in range(4):
                vv = tailb[pl.ds(v * 64 + 16 * k, 16)]
                plsc.store_scatter(outbs[0], [v * D_MODEL + 16 * k + lane],
                                   vv)
            return c

        lax.fori_loop(0, 64, vrow, 0)
        pltpu.sync_copy(outbs[0].at[pl.ds(0, 64 * P1S)],
                        out_hbm.at[pl.ds((N_TC - 1) * (128 * P1S),
                                         64 * P1S)])


def _emb_body(xt_hbm, lut_hbm, out_hbm, idxs, rowss, tbufs,
              isems, gsems, wsems):
    wid = lax.axis_index("s") * 2 + lax.axis_index("c")
    lane = lax.iota(jnp.int32, 16)
    dcols = [lane + 16 * k for k in range(4)]

    def idx_start(t, b):
        pltpu.async_copy(xt_hbm.at[t // 2, 2 * wid + t % 2],
                         idxs[b], isems[b])

    def idx_wait(b):
        pltpu.make_async_copy(xt_hbm.at[0, 0], idxs[b], isems[b]).wait()

    def gather_start(b):
        pltpu.async_copy(lut_hbm.at[idxs[b].at[0]],
                         rowss[b].at[pl.ds(0, 128)], gsems[b])
        pltpu.async_copy(lut_hbm.at[idxs[b].at[1]],
                         rowss[b].at[pl.ds(128, 128)], gsems[b])

    def gather_wait(b):
        for _ in range(2):
            pltpu.make_async_copy(lut_hbm.at[idxs[b].at[0]],
                                  rowss[b].at[pl.ds(0, 128)],
                                  gsems[b]).wait()

    def transpose(br, bt):
        rows, tb = rowss[br], tbufs[bt]

        @plsc.parallel_loop(0, P_B, step=1, unroll=2)
        def _(r):
            rb = jnp.full((16,), r, jnp.int32)
            for k in range(4):
                v = rows[r, pl.ds(16 * k, 16)] * SCALE
                plsc.store_scatter(tb, [dcols[k], rb], v)

    def write_start(t, bt):
        c = 2 * wid + t % 2
        pltpu.async_copy(tbufs[bt].at[:, pl.ds(0, P_B)],
                         out_hbm.at[t // 2, :, pl.ds(c * P_B, P_B)],
                         wsems[bt])

    def write_wait(bt):
        pltpu.make_async_copy(tbufs[bt].at[:, pl.ds(0, P_B)],
                              out_hbm.at[0, :, pl.ds(0, P_B)],
                              wsems[bt]).wait()

    def item(t, b, first, last):
        br = b % NR
        bt = b % NT
        gather_wait(br)
        if not last:
            idx_start(t + NR, br)
        if not first:
            write_wait(bt)
        transpose(br, bt)
        if not last:
            idx_wait(br)
            gather_start(br)
        write_start(t, bt)

    for b in range(NR):
        idx_start(b, b)
    for b in range(NR):
        idx_wait(b)
        gather_start(b)

    for t in range(NR):
        item(t, t, first=t < NT, last=False)

    def group(g, carry):
        for b in range(NR):
            item(g * NR + b, b, first=False, last=False)
        return carry

    lax.fori_loop(1, N_ITEMS // NR - 1, group, 0)

    for t in range(N_ITEMS - NR, N_ITEMS):
        item(t, t % NR, first=False, last=True)
    for bt in range(NT):
        write_wait(bt)


def kernel(x, lut):
    B, S = x.shape
    mesh = plsc.VectorSubcoreMesh(core_axis_name="c", subcore_axis_name="s")

    lutp = pl.kernel(
        _tr_body,
        mesh=mesh,
        out_type=jax.ShapeDtypeStruct((N_TC * 128 * P1S,), jnp.float32),
        scratch_types=[
            [pltpu.VMEM((8, 8, 128), jnp.float32) for _ in range(NB1)],
            [pltpu.VMEM((128 * P1S,), jnp.float32) for _ in range(NO1)],
            pltpu.VMEM((64 * D_MODEL,), jnp.float32),
            [pltpu.SemaphoreType.DMA for _ in range(NB1)],
            [pltpu.SemaphoreType.DMA for _ in range(NO1)],
        ],
        compiler_params=pltpu.CompilerParams(
            use_tc_tiling_on_sc=True, needs_layout_passes=False),
    )(lut.T, lut[999936:].reshape(64 * D_MODEL))

    xt = x.T.astype(jnp.int32).reshape(S, N_C, 2, 128)
    out = pl.kernel(
        _emb_body,
        mesh=mesh,
        out_type=jax.ShapeDtypeStruct((SEQ, D_MODEL, N_B), jnp.float32),
        scratch_types=[
            [pltpu.VMEM((2, 128), jnp.int32) for _ in range(NR)],
            [pltpu.VMEM((P_B, D_MODEL), jnp.float32) for _ in range(NR)],
            [pltpu.VMEM((D_MODEL, P_B + 1), jnp.float32) for _ in range(NT)],
            [pltpu.SemaphoreType.DMA for _ in range(NR)],
            [pltpu.SemaphoreType.DMA for _ in range(NR)],
            [pltpu.SemaphoreType.DMA for _ in range(NT)],
        ],
        compiler_params=pltpu.CompilerParams(
            use_tc_tiling_on_sc=False, needs_layout_passes=False),
    )(xt, lutp.reshape(N_TC * 128, D_MODEL))
    return jnp.transpose(out, (2, 0, 1))
